# speedup vs baseline: 4.6522x; 4.6522x over previous
import functools

import jax
import jax.numpy as jnp
from jax import lax
from jax.experimental import pallas as pl
from jax.experimental.pallas import tpu as pltpu
from jax.experimental.pallas import tpu_sc as plsc

NC = 2
NS = 16
NW = NC * NS
CHUNK = 128


def _sc_aggregate(features, src_t, dst_t, n_pad, ch):
    D = features.shape[1]
    rpt = n_pad // NS
    assert rpt % CHUNK == 0

    mesh = plsc.VectorSubcoreMesh(core_axis_name="c", subcore_axis_name="s")

    @functools.partial(
        pl.kernel,
        mesh=mesh,
        out_type=jax.ShapeDtypeStruct((NC, n_pad, D), jnp.float32),
        scratch_types=[
            pltpu.VMEM((ch, CHUNK), jnp.int32),
            pltpu.VMEM((ch, CHUNK), jnp.int32),
            pltpu.VMEM((CHUNK, D), jnp.float32),
            pltpu.VMEM_SHARED((n_pad, D), jnp.float32),
            pltpu.SemaphoreType.DMA,
        ],
    )
    def agg(feat_hbm, src_hbm, dst_hbm, out_hbm, src_v, dst_v, gbuf, acc, sem):
        c = lax.axis_index("c")
        s = lax.axis_index("s")
        wid = s * NC + c
        base = s * rpt

        zero = jnp.zeros((16,), jnp.float32)

        def zrow(r, carry):
            for j in range(D // 16):
                gbuf[r, pl.ds(j * 16, 16)] = zero
            return carry

        lax.fori_loop(0, CHUNK, zrow, 0)
        for k in range(rpt // CHUNK):
            pltpu.sync_copy(gbuf, acc.at[pl.ds(base + k * CHUNK, CHUNK)])

        pltpu.sync_copy(src_hbm.at[wid], src_v)
        pltpu.sync_copy(dst_hbm.at[wid], dst_v)
        plsc.subcore_barrier()

        def body(j, carry):
            pltpu.async_copy(feat_hbm.at[src_v.at[j]], gbuf, sem).wait()
            pltpu.sync_copy(gbuf, acc.at[dst_v.at[j]], add=True)
            return carry

        lax.fori_loop(0, ch, body, 0)
        plsc.subcore_barrier()

        pltpu.sync_copy(acc.at[pl.ds(base, rpt)],
                        out_hbm.at[c].at[pl.ds(base, rpt)])

    return agg(features, src_t, dst_t)


def _tc_transform(p0, p1, features, W1, W2):
    n, D = features.shape
    outd = W1.shape[1]
    blk = 1000
    assert n % blk == 0

    def body(p0_ref, p1_ref, f_ref, w1_ref, w2_ref, o_ref):
        ns = p0_ref[...] + p1_ref[...]
        f = f_ref[...]
        acc = jnp.dot(ns + f, w1_ref[...], preferred_element_type=jnp.float32)
        acc += jnp.dot(ns * f, w2_ref[...], preferred_element_type=jnp.float32)
        o_ref[...] = jnp.where(acc >= 0, acc, 0.2 * acc)

    row_spec = pl.BlockSpec((blk, D), lambda i: (i, 0))
    w_spec = pl.BlockSpec((D, outd), lambda i: (0, 0))
    return pl.pallas_call(
        body,
        grid=(n // blk,),
        in_specs=[row_spec, row_spec, row_spec, w_spec, w_spec],
        out_specs=pl.BlockSpec((blk, outd), lambda i: (i, 0)),
        out_shape=jax.ShapeDtypeStruct((n, outd), jnp.float32),
    )(p0, p1, features, W1, W2)


def kernel(features, edge_index, W1, W2):
    n, D = features.shape
    E = edge_index.shape[1]
    ch = pl.cdiv(E, NW * CHUNK)
    e_pad = NW * ch * CHUNK
    n_pad = ((n + 1 + NS * CHUNK - 1) // (NS * CHUNK)) * (NS * CHUNK)

    src = edge_index[0]
    dst = edge_index[1]
    pad = e_pad - E
    if pad:
        src = jnp.concatenate([src, jnp.zeros((pad,), jnp.int32)])
        dst = jnp.concatenate([dst, jnp.full((pad,), n, jnp.int32)])
    src_t = src.reshape(NW, ch, CHUNK)
    dst_t = dst.reshape(NW, ch, CHUNK)

    partials = _sc_aggregate(features, src_t, dst_t, n_pad, ch)
    return _tc_transform(partials[0, :n], partials[1, :n], features, W1, W2)
